# baseline (speedup 1.0000x reference)
import jax
import jax.numpy as jnp
from jax.experimental import pallas as pl
from jax.experimental.pallas import tpu as pltpu
from jax.experimental.pallas import tpu_sc as plsc

_WINDOW = 128


def kernel(token_id, emb_norm):
    num_indices = token_id.size
    embed_dim = emb_norm.shape[1]
    flat = token_id.reshape(1, num_indices).astype(jnp.int32)

    mesh = plsc.VectorSubcoreMesh(core_axis_name="c", subcore_axis_name="s")

    @pl.kernel(
        out_type=jax.ShapeDtypeStruct((num_indices, embed_dim), emb_norm.dtype),
        mesh=mesh,
        compiler_params=pltpu.CompilerParams(use_tc_tiling_on_sc=False),
    )
    def embed_gather(x_hbm, i_hbm, o_hbm):
        def body(i_vmem, o_vmem):
            pltpu.sync_copy(x_hbm.at[i_vmem.at[0]], o_vmem)

        pltpu.emit_pipeline(
            body,
            grid=(num_indices // _WINDOW,),
            in_specs=[
                pl.BlockSpec((1, _WINDOW), index_map=lambda i: (0, i)),
            ],
            out_specs=[
                pl.BlockSpec((_WINDOW, embed_dim), index_map=lambda i: (i, 0)),
            ],
            core_axis_name=("c", "s"),
            dimension_semantics=(pltpu.PARALLEL,),
        )(i_hbm, o_hbm)

    return embed_gather(emb_norm, flat)


# speedup vs baseline: 1.0440x; 1.0440x over previous
import jax
import jax.numpy as jnp
from jax.experimental import pallas as pl
from jax.experimental.pallas import tpu as pltpu
from jax.experimental.pallas import tpu_sc as plsc

_WINDOW = 128

_VOCAB = 1000000
_DIM = 64
_N_IDX = 819200

_A_BLK = 4096
_A_GRID = 123
_HALF = _A_BLK * _A_GRID
_C_BLK = 6400
_C_GRID = _N_IDX // _C_BLK


def _relayout_table(t_view):
    def body(x_lo, x_hi, o_ref):
        x = jnp.concatenate([x_lo[...], x_hi[...]], axis=0)
        o_ref[...] = jnp.swapaxes(x, 0, 1)

    return pl.pallas_call(
        body,
        grid=(_A_GRID,),
        in_specs=[
            pl.BlockSpec((_DIM, _A_BLK), lambda c: (0, c)),
            pl.BlockSpec(
                (_DIM, _A_BLK),
                lambda c: (0, jnp.minimum(c + _A_GRID, _VOCAB // _A_BLK)),
            ),
        ],
        out_specs=pl.BlockSpec((_A_BLK, 128), lambda c: (c, 0)),
        out_shape=jax.ShapeDtypeStruct((_HALF, 128), jnp.float32),
    )(t_view, t_view)


def _sc_gather(table_lin, flat_idx):
    mesh = plsc.VectorSubcoreMesh(core_axis_name="c", subcore_axis_name="s")

    @pl.kernel(
        out_type=jax.ShapeDtypeStruct((_N_IDX, _DIM), jnp.float32),
        mesh=mesh,
        compiler_params=pltpu.CompilerParams(use_tc_tiling_on_sc=False),
    )
    def embed_gather(x_hbm, i_hbm, o_hbm):
        def body(i_vmem, o_vmem):
            pltpu.sync_copy(x_hbm.at[i_vmem.at[0]], o_vmem)

        pltpu.emit_pipeline(
            body,
            grid=(_N_IDX // _WINDOW,),
            in_specs=[
                pl.BlockSpec((1, _WINDOW), index_map=lambda i: (0, i)),
            ],
            out_specs=[
                pl.BlockSpec((_WINDOW, _DIM), index_map=lambda i: (i, 0)),
            ],
            core_axis_name=("c", "s"),
            dimension_semantics=(pltpu.PARALLEL,),
        )(i_hbm, o_hbm)

    return embed_gather(table_lin, flat_idx)


def _relayout_out(out_view):
    def body(x_ref, o_ref):
        y = jnp.swapaxes(x_ref[...], 0, 1)
        o_ref[...] = jnp.concatenate([y[0:64, :], y[64:128, :]], axis=1)

    return pl.pallas_call(
        body,
        grid=(_C_GRID,),
        in_specs=[pl.BlockSpec((_C_BLK // 2, 128), lambda r: (r, 0))],
        out_specs=pl.BlockSpec((64, _C_BLK), lambda r: (0, r)),
        out_shape=jax.ShapeDtypeStruct((64, _N_IDX), jnp.float32),
    )(out_view)


def kernel(token_id, emb_norm):
    flat = token_id.reshape(-1).astype(jnp.int32)
    t_o = jnp.transpose(flat.reshape(_C_GRID, 2, _C_BLK // 2), (0, 2, 1))
    t_o = t_o.reshape(1, _N_IDX)
    idx_sc = jnp.where(t_o < _HALF, 2 * t_o, 2 * t_o - (2 * _HALF - 1))

    table_folded = _relayout_table(emb_norm.T)
    table_lin = table_folded.reshape(2 * _HALF, _DIM)
    gathered = _sc_gather(table_lin, idx_sc)
    out_t = _relayout_out(gathered.reshape(_N_IDX // 2, 128))
    return out_t.T


# speedup vs baseline: 2.0296x; 1.9440x over previous
import jax
import jax.numpy as jnp
from jax.experimental import pallas as pl
from jax.experimental.pallas import tpu as pltpu
from jax.experimental.pallas import tpu_sc as plsc

_WINDOW = 128

_VOCAB = 1000000
_DIM = 64
_N_IDX = 819200

_A_BLK = 4096
_A_GRID = 123
_HALF = _A_BLK * _A_GRID
_C_BLK = 6400
_C_GRID = _N_IDX // _C_BLK

_N_WINDOWS = _N_IDX // _WINDOW
_WPC = _C_BLK // _WINDOW


def _relayout_table(t_view):
    def body(x_lo, x_hi, o_ref):
        x = jnp.concatenate([x_lo[...], x_hi[...]], axis=0)
        o_ref[...] = jnp.swapaxes(x, 0, 1)

    return pl.pallas_call(
        body,
        grid=(_A_GRID,),
        in_specs=[
            pl.BlockSpec((_DIM, _A_BLK), lambda c: (0, c)),
            pl.BlockSpec(
                (_DIM, _A_BLK),
                lambda c: (0, jnp.minimum(c + _A_GRID, _VOCAB // _A_BLK)),
            ),
        ],
        out_specs=pl.BlockSpec((_A_BLK, 128), lambda c: (c, 0)),
        out_shape=jax.ShapeDtypeStruct((_HALF, 128), jnp.float32),
    )(t_view, t_view)


def _remap(v):
    t2 = v + v
    return jnp.where(v < _HALF, t2, t2 - (2 * _HALF - 1))


def _sc_gather(table_lin, idx_nat):
    mesh = plsc.VectorSubcoreMesh(core_axis_name="c", subcore_axis_name="s")

    @pl.kernel(
        out_type=jax.ShapeDtypeStruct((_N_IDX, _DIM), jnp.float32),
        mesh=mesh,
        scratch_types=[pltpu.VMEM((_WINDOW,), jnp.int32)],
        compiler_params=pltpu.CompilerParams(
            use_tc_tiling_on_sc=False, needs_layout_passes=False
        ),
    )
    def embed_gather(x_hbm, i_hbm, o_hbm, scr):
        lane = jax.lax.iota(jnp.int32, 16)

        def body(i_a, i_b, o_vmem):
            for n in range(4):
                a = _remap(i_a[0, pl.ds(16 * n, 16)])
                plsc.store_scatter(scr, [2 * (lane + 16 * n)], a)
                b = _remap(i_b[0, pl.ds(16 * n, 16)])
                plsc.store_scatter(scr, [2 * (lane + 16 * n) + 1], b)
            pltpu.sync_copy(x_hbm.at[scr], o_vmem)

        def amap(i):
            u = 100 * (i // _WPC) + i % _WPC
            return (u // 2, u % 2)

        def bmap(i):
            u = 100 * (i // _WPC) + i % _WPC + _WPC
            return (u // 2, u % 2)

        pltpu.emit_pipeline(
            body,
            grid=(_N_WINDOWS,),
            in_specs=[
                pl.BlockSpec((1, 64), index_map=amap),
                pl.BlockSpec((1, 64), index_map=bmap),
            ],
            out_specs=[
                pl.BlockSpec((_WINDOW, _DIM), index_map=lambda i: (i, 0)),
            ],
            core_axis_name=("c", "s"),
            dimension_semantics=(pltpu.PARALLEL,),
        )(i_hbm, i_hbm, o_hbm)

    return embed_gather(table_lin, idx_nat)


def _relayout_out(out_view):
    def body(x_ref, o_ref):
        y = jnp.swapaxes(x_ref[...], 0, 1)
        o_ref[...] = jnp.concatenate([y[0:64, :], y[64:128, :]], axis=1)

    return pl.pallas_call(
        body,
        grid=(_C_GRID,),
        in_specs=[pl.BlockSpec((_C_BLK // 2, 128), lambda r: (r, 0))],
        out_specs=pl.BlockSpec((64, _C_BLK), lambda r: (0, r)),
        out_shape=jax.ShapeDtypeStruct((64, _N_IDX), jnp.float32),
    )(out_view)


def kernel(token_id, emb_norm):
    idx_nat = token_id.reshape(_N_WINDOWS, _WINDOW).astype(jnp.int32)
    table_folded = _relayout_table(emb_norm.T)
    table_lin = table_folded.reshape(2 * _HALF, _DIM)
    gathered = _sc_gather(table_lin, idx_nat)
    out_t = _relayout_out(gathered.reshape(_N_IDX // 2, 128))
    return out_t.T


# speedup vs baseline: 2.3921x; 1.1786x over previous
import jax
import jax.numpy as jnp
from jax.experimental import pallas as pl
from jax.experimental.pallas import tpu as pltpu
from jax.experimental.pallas import tpu_sc as plsc

_WINDOW = 128

_VOCAB = 1000000
_DIM = 64
_N_IDX = 819200

_A_BLK = 4096
_A_GRID = 123
_HALF = _A_BLK * _A_GRID
_C_BLK = 6400
_C_GRID = _N_IDX // _C_BLK

_N_WINDOWS = _N_IDX // _WINDOW
_WPC = _C_BLK // _WINDOW


def _relayout_table(t_view):
    def body(x_lo, x_hi, o_ref):
        x = jnp.concatenate([x_lo[...], x_hi[...]], axis=0)
        o_ref[...] = jnp.swapaxes(x, 0, 1)

    return pl.pallas_call(
        body,
        grid=(_A_GRID,),
        in_specs=[
            pl.BlockSpec((_DIM, _A_BLK), lambda c: (0, c)),
            pl.BlockSpec(
                (_DIM, _A_BLK),
                lambda c: (0, jnp.minimum(c + _A_GRID, _VOCAB // _A_BLK)),
            ),
        ],
        out_specs=pl.BlockSpec((_A_BLK, 128), lambda c: (c, 0)),
        out_shape=jax.ShapeDtypeStruct((_HALF, 128), jnp.float32),
        compiler_params=pltpu.CompilerParams(
            dimension_semantics=("parallel",)
        ),
    )(t_view, t_view)


def _remap(v):
    t2 = v + v
    return jnp.where(v < _HALF, t2, t2 - (2 * _HALF - 1))


def _sc_gather(table_lin, idx_nat):
    mesh = plsc.VectorSubcoreMesh(core_axis_name="c", subcore_axis_name="s")

    @pl.kernel(
        out_type=jax.ShapeDtypeStruct((_N_IDX, _DIM), jnp.float32),
        mesh=mesh,
        scratch_types=[pltpu.VMEM((_WINDOW,), jnp.int32)],
        compiler_params=pltpu.CompilerParams(
            use_tc_tiling_on_sc=False, needs_layout_passes=False
        ),
    )
    def embed_gather(x_hbm, i_hbm, o_hbm, scr):
        lane = jax.lax.iota(jnp.int32, 16)

        def body(i_a, i_b, o_vmem):
            for n in range(4):
                a = _remap(i_a[0, pl.ds(16 * n, 16)])
                plsc.store_scatter(scr, [2 * (lane + 16 * n)], a)
                b = _remap(i_b[0, pl.ds(16 * n, 16)])
                plsc.store_scatter(scr, [2 * (lane + 16 * n) + 1], b)
            pltpu.sync_copy(x_hbm.at[scr], o_vmem)

        def amap(i):
            u = 100 * (i // _WPC) + i % _WPC
            return (u // 2, u % 2)

        def bmap(i):
            u = 100 * (i // _WPC) + i % _WPC + _WPC
            return (u // 2, u % 2)

        pltpu.emit_pipeline(
            body,
            grid=(_N_WINDOWS,),
            in_specs=[
                pl.BlockSpec((1, 64), index_map=amap),
                pl.BlockSpec((1, 64), index_map=bmap),
            ],
            out_specs=[
                pl.BlockSpec((_WINDOW, _DIM), index_map=lambda i: (i, 0)),
            ],
            core_axis_name=("c", "s"),
            dimension_semantics=(pltpu.PARALLEL,),
        )(i_hbm, i_hbm, o_hbm)

    return embed_gather(table_lin, idx_nat)


def _relayout_out(out_view):
    def body(x_ref, o_ref):
        y = jnp.swapaxes(x_ref[...], 0, 1)
        o_ref[...] = jnp.concatenate([y[0:64, :], y[64:128, :]], axis=1)

    return pl.pallas_call(
        body,
        grid=(_C_GRID,),
        in_specs=[pl.BlockSpec((_C_BLK // 2, 128), lambda r: (r, 0))],
        out_specs=pl.BlockSpec((64, _C_BLK), lambda r: (0, r)),
        out_shape=jax.ShapeDtypeStruct((64, _N_IDX), jnp.float32),
        compiler_params=pltpu.CompilerParams(
            dimension_semantics=("parallel",)
        ),
    )(out_view)


def kernel(token_id, emb_norm):
    idx_nat = token_id.reshape(_N_WINDOWS, _WINDOW).astype(jnp.int32)
    table_folded = _relayout_table(emb_norm.T)
    table_lin = table_folded.reshape(2 * _HALF, _DIM)
    gathered = _sc_gather(table_lin, idx_nat)
    out_t = _relayout_out(gathered.reshape(_N_IDX // 2, 128))
    return out_t.T
